# Initial kernel scaffold; baseline (speedup 1.0000x reference)
import jax
import jax.numpy as jnp
from jax.experimental import pallas as pl


def kernel(condition, hx, embed_task, Wih_f, Whh_f, bih_f, bhh_f, Wih_b, Whh_b, bih_b, bhh_b, Wih_c, Whh_c, bih_c, bhh_c, W0, b0, W1, b1, W2, b2, Wc, bc, Wa, ba, lines, active, actions):
    raise NotImplementedError("write your pallas kernel here")


# speedup vs baseline: 4.9132x; 4.9132x over previous
import jax
import jax.numpy as jnp
from jax import lax
from jax.experimental import pallas as pl
from jax.experimental.pallas import tpu as pltpu

T, N, L, H, COND, NA, NL = 64, 1024, 16, 256, 64, 16, 64
HX = 4 + H + 2 * NA
H3 = 3 * H

ENC_BN = 128
DEC_BN = 256
NT = 4
NC = 1


def _gru_math(gi, gh_rz, hn, h):
    r = jax.nn.sigmoid(gi[:, :H] + gh_rz[:, :H])
    z = jax.nn.sigmoid(gi[:, H:2 * H] + gh_rz[:, H:2 * H])
    n = jnp.tanh(gi[:, 2 * H:] + r * hn)
    return (1.0 - z) * n + z * h


def _enc_kernel(linesT_ref, emb_ref, wihT_f_ref, whhT_f_ref, bih_f_ref,
                bhh_f_ref, wihT_b_ref, whhT_b_ref, bih_b_ref, bhh_b_ref,
                out_ref, gi2_f_ref, gi2_b_ref, hf_ref, hb_ref):
    BN = ENC_BN
    M = L * BN
    linesT = linesT_ref[...]
    oh = (linesT[:, :, None]
          == lax.broadcasted_iota(jnp.int32, (L, BN, NL), 2)).astype(jnp.float32)
    emb = emb_ref[...]
    m2 = jnp.dot(oh.reshape(M, NL), emb,
                 preferred_element_type=jnp.float32)

    for gi2_ref, wihT_ref, bih_ref, bhh_ref in (
            (gi2_f_ref, wihT_f_ref, bih_f_ref, bhh_f_ref),
            (gi2_b_ref, wihT_b_ref, bih_b_ref, bhh_b_ref)):
        bhh = bhh_ref[...]
        badd = bih_ref[...] + jnp.concatenate(
            [bhh[:, :2 * H], jnp.zeros((1, H), jnp.float32)], axis=1)
        gi = jnp.dot(m2, wihT_ref[...],
                     preferred_element_type=jnp.float32) + badd
        gi3 = gi.reshape(L, BN, H3)
        gi2_ref[0:L] = gi3
        gi2_ref[L:2 * L] = gi3

    whhT_f = whhT_f_ref[...]
    whhT_b = whhT_b_ref[...]
    bhh_n_f = bhh_f_ref[...][:, 2 * H:]
    bhh_n_b = bhh_b_ref[...][:, 2 * H:]

    hf_ref[...] = jnp.zeros((M, H), jnp.float32)
    hb_ref[...] = jnp.zeros((M, H), jnp.float32)

    def enc_step(t, carry):
        xf = gi2_f_ref[pl.ds(t, L)].reshape(M, H3)
        hf = hf_ref[...]
        ghf = jnp.dot(hf, whhT_f, preferred_element_type=jnp.float32)
        hf_ref[...] = _gru_math(xf, ghf[:, :2 * H],
                                ghf[:, 2 * H:] + bhh_n_f, hf)
        xb = gi2_b_ref[pl.ds(L - 1 - t, L)].reshape(M, H3)
        hb = hb_ref[...]
        ghb = jnp.dot(hb, whhT_b, preferred_element_type=jnp.float32)
        hb_ref[...] = _gru_math(xb, ghb[:, :2 * H],
                                ghb[:, 2 * H:] + bhh_n_b, hb)
        return carry

    lax.fori_loop(0, L, enc_step, 0)

    out_ref[:, :, 0:H] = hf_ref[...].reshape(L, BN, H)
    out_ref[:, :, H:2 * H] = hb_ref[...].reshape(L, BN, H)


def _dec_kernel(g_ref, cond_ref, scal_ref, noise_ref,
                hx_ref, wccT_ref, wcgT_ref, whhT_ref, bih_ref, bhh_ref,
                w0T_ref, b0_ref, w1T_ref, b1_ref, w2T_ref, b2_ref,
                wacT_ref, bac_ref, out_ref, h_ref):
    wccT = wccT_ref[...]
    wcgT = wcgT_ref[...]
    whhT = whhT_ref[...]
    bih = bih_ref[...]
    bhh = bhh_ref[...]
    badd = bih + jnp.concatenate(
        [bhh[:, :2 * H], jnp.zeros((1, H), jnp.float32)], axis=1)
    bhh_n = bhh[:, 2 * H:]
    w0T, b0 = w0T_ref[...], b0_ref[...]
    w1T, b1 = w1T_ref[...], b1_ref[...]
    w2T, b2 = w2T_ref[...], b2_ref[...]
    wacT, bac = wacT_ref[...], bac_ref[...]
    p0 = hx_ref[:, 4 + H + NA:4 + H + 2 * NA]

    @pl.when(pl.program_id(2) == 0)
    def _():
        h_ref[...] = hx_ref[:, 4:4 + H]
    h0 = h_ref[...]
    lrow = lax.broadcasted_iota(jnp.int32, (1, L), 1).astype(jnp.float32)
    narange = lax.broadcasted_iota(jnp.int32, (1, NA), 1).astype(jnp.float32)

    def step(t, h):
        sc = scal_ref[t]
        a0_col, a1_col, w_col = sc[:, 0:1], sc[:, 1:2], sc[:, 2:3]
        m2 = jnp.where(w_col == lrow, 1.0, 0.0)
        g = m2[:, 0:1] * g_ref[0]
        for l in range(1, L):
            g = g + m2[:, l:l + 1] * g_ref[l]
        c = cond_ref[t]
        gi = (jnp.dot(c, wccT, preferred_element_type=jnp.float32)
              + jnp.dot(g, wcgT, preferred_element_type=jnp.float32) + badd)
        gh = jnp.dot(h, whhT, preferred_element_type=jnp.float32)
        h = _gru_math(gi, gh[:, :2 * H], gh[:, 2 * H:] + bhh_n, h)
        z = jax.nn.relu(jnp.dot(h, w0T, preferred_element_type=jnp.float32) + b0)
        z = jax.nn.relu(jnp.dot(z, w1T, preferred_element_type=jnp.float32) + b1)
        z = jax.nn.relu(jnp.dot(z, w2T, preferred_element_type=jnp.float32) + b2)
        lv = jnp.dot(z, wacT, preferred_element_type=jnp.float32) + bac
        logits = lv[:, :NA]
        mx = jnp.max(logits, axis=-1, keepdims=True)
        e = jnp.exp(logits - mx)
        probs = e / jnp.sum(e, axis=-1, keepdims=True)
        y = logits + noise_ref[t]
        ymx = jnp.max(y, axis=-1, keepdims=True)
        cand = jnp.where(y == ymx, narange, float(NA))
        samp_col = jnp.min(cand, axis=-1, keepdims=True)
        a_col = jnp.where(a0_col < 0.0, samp_col, a0_col)
        s4 = jnp.concatenate(
            [a_col, a1_col, w_col, lv[:, NA:NA + 1]], axis=-1)
        out_ref[t] = jnp.concatenate([s4, h, probs, p0], axis=-1)
        return h

    h_ref[...] = lax.fori_loop(0, T // NT, step, h0)


def _encode(lines, embed_task, Wih_f, Whh_f, bih_f, bhh_f,
            Wih_b, Whh_b, bih_b, bhh_b, interpret=False):
    nbc = N // ENC_BN // NC
    blk = lambda c, j: c * nbc + j
    full = lambda shape: pl.BlockSpec(shape, lambda c, j: tuple(0 for _ in shape))
    return pl.pallas_call(
        _enc_kernel,
        out_shape=jax.ShapeDtypeStruct((L, N, 2 * H), jnp.float32),
        grid=(NC, nbc),
        in_specs=[
            pl.BlockSpec((L, ENC_BN), lambda c, j: (0, blk(c, j))),
            full((NL, H)),
            full((H, H3)), full((H, H3)), full((1, H3)), full((1, H3)),
            full((H, H3)), full((H, H3)), full((1, H3)), full((1, H3)),
        ],
        out_specs=pl.BlockSpec((L, ENC_BN, 2 * H),
                               lambda c, j: (0, blk(c, j), 0)),
        scratch_shapes=[
            pltpu.VMEM((2 * L, ENC_BN, H3), jnp.float32),
            pltpu.VMEM((2 * L, ENC_BN, H3), jnp.float32),
            pltpu.VMEM((L * ENC_BN, H), jnp.float32),
            pltpu.VMEM((L * ENC_BN, H), jnp.float32),
        ],
        compiler_params=pltpu.CompilerParams(
            dimension_semantics=("arbitrary", "arbitrary"),
            vmem_limit_bytes=100 * 1024 * 1024,
        ),
        name="recur_encoder",
        interpret=interpret,
    )(lines.T, embed_task,
      Wih_f.T, Whh_f.T, bih_f.reshape(1, H3), bhh_f.reshape(1, H3),
      Wih_b.T, Whh_b.T, bih_b.reshape(1, H3), bhh_b.reshape(1, H3))


def _decode(G, condition, active, a0, a1, noise, hxs,
            Wih_c, Whh_c, bih_c, bhh_c, W0, b0, W1, b1, W2, b2,
            Wc, bc, Wa, ba, interpret=False):
    nbc = N // DEC_BN // NC
    TQ = T // NT
    blk = lambda c, j, q: c * nbc + j
    full = lambda shape: pl.BlockSpec(
        shape, lambda c, j, q: tuple(0 for _ in shape))
    wac = jnp.concatenate([Wa, Wc], axis=0)
    bac = jnp.concatenate([ba, bc]).reshape(1, NA + 1)
    return pl.pallas_call(
        _dec_kernel,
        out_shape=jax.ShapeDtypeStruct((T, N, HX), jnp.float32),
        grid=(NC, nbc, NT),
        in_specs=[
            pl.BlockSpec((L, DEC_BN, 2 * H), lambda c, j, q: (0, blk(c, j, q), 0)),
            pl.BlockSpec((TQ, DEC_BN, COND), lambda c, j, q: (q, blk(c, j, q), 0)),
            pl.BlockSpec((TQ, DEC_BN, 3), lambda c, j, q: (q, blk(c, j, q), 0)),
            pl.BlockSpec((TQ, DEC_BN, NA), lambda c, j, q: (q, blk(c, j, q), 0)),
            pl.BlockSpec((DEC_BN, HX), lambda c, j, q: (blk(c, j, q), 0)),
            full((COND, H3)), full((2 * H, H3)), full((H, H3)),
            full((1, H3)), full((1, H3)),
            full((H, H)), full((1, H)), full((H, H)), full((1, H)),
            full((H, H)), full((1, H)),
            full((H, NA + 1)), full((1, NA + 1)),
        ],
        out_specs=pl.BlockSpec((TQ, DEC_BN, HX),
                               lambda c, j, q: (q, blk(c, j, q), 0)),
        scratch_shapes=[pltpu.VMEM((DEC_BN, H), jnp.float32)],
        compiler_params=pltpu.CompilerParams(
            dimension_semantics=("arbitrary", "arbitrary", "arbitrary"),
            vmem_limit_bytes=100 * 1024 * 1024,
        ),
        name="recur_decoder",
        interpret=interpret,
    )(G, condition,
      jnp.stack([a0, a1, active], axis=-1).astype(jnp.float32),
      noise, hxs,
      Wih_c[:, :COND].T, Wih_c[:, COND:].T, Whh_c.T,
      bih_c.reshape(1, H3), bhh_c.reshape(1, H3),
      W0.T, b0.reshape(1, H), W1.T, b1.reshape(1, H),
      W2.T, b2.reshape(1, H), wac.T, bac)


def kernel(condition, hx, embed_task, Wih_f, Whh_f, bih_f, bhh_f,
           Wih_b, Whh_b, bih_b, bhh_b, Wih_c, Whh_c, bih_c, bhh_c,
           W0, b0, W1, b1, W2, b2, Wc, bc, Wa, ba, lines, active, actions,
           interpret=False):
    base_key = jax.random.key(1)
    keys = jax.vmap(lambda t: jax.random.fold_in(base_key, t))(jnp.arange(T))
    noise = jax.vmap(
        lambda k: jax.random.gumbel(k, (N, NA), jnp.float32))(keys)

    G = _encode(lines, embed_task, Wih_f, Whh_f, bih_f, bhh_f,
                Wih_b, Whh_b, bih_b, bhh_b, interpret=interpret)
    return _decode(G, condition, active, actions[:, :, 0], actions[:, :, 1],
                   noise, hx[0], Wih_c, Whh_c, bih_c, bhh_c,
                   W0, b0, W1, b1, W2, b2, Wc, bc, Wa, ba,
                   interpret=interpret)
